# Initial kernel scaffold; baseline (speedup 1.0000x reference)
import jax
import jax.numpy as jnp
from jax.experimental import pallas as pl


def kernel(edge_index, edge_values, user_n_j, item_n_j, user_emb, item_emb, mean_weight):
    raise NotImplementedError("write your pallas kernel here")


# speedup vs baseline: 4.1882x; 4.1882x over previous
import functools

import jax
import jax.numpy as jnp
from jax import lax
from jax.experimental import pallas as pl
from jax.experimental.pallas import tpu as pltpu
from jax.experimental.pallas import tpu_sc as plsc

N_U = 10000
N_I = 10000
E_TOT = 160000
D = 256
DH = 128
N_TILES = 16
CHUNK = 80
GCH = 25
NGROUP = 5
EDGES_PER_TILE = CHUNK * GCH * NGROUP
N_PAD = 10240
ROWS_PER_TILE = N_PAD // N_TILES
RB = 1000


def _spmm_body(dst_hbm, src_hbm, ev_hbm, item0_hbm, item1_hbm, out_hbm,
               dst_v, src_v, ev_v, rows_v, acc):
    c = lax.axis_index("c")
    s = lax.axis_index("s")
    rbase = s * ROWS_PER_TILE

    zero16 = jnp.zeros((16,), jnp.float32)

    def _zb(i, carry):
        for j in range(DH // 16):
            rows_v[i, pl.ds(j * 16, 16)] = zero16
        return carry

    lax.fori_loop(0, CHUNK, _zb, 0)
    for r in range(ROWS_PER_TILE // CHUNK):
        pltpu.sync_copy(rows_v, acc.at[pl.ds(rbase + r * CHUNK, CHUNK)])
    plsc.subcore_barrier()

    def _run(item_hbm):
        for g in range(NGROUP):
            pltpu.sync_copy(dst_hbm.at[s, g], dst_v)
            pltpu.sync_copy(src_hbm.at[s, g], src_v)
            pltpu.sync_copy(ev_hbm.at[s, g], ev_v)

            def _chunk(j, carry):
                pltpu.sync_copy(item_hbm.at[dst_v.at[j]], rows_v)

                def _edge16(gg, carry2):
                    evv = ev_v[j, pl.ds(gg * 16, 16)]
                    for lane in range(16):
                        val = evv[lane]
                        e = gg * 16 + lane
                        for k in range(DH // 16):
                            sl = pl.ds(k * 16, 16)
                            rows_v[e, sl] = rows_v[e, sl] * val
                    return carry2

                lax.fori_loop(0, CHUNK // 16, _edge16, 0)

                pltpu.sync_copy(rows_v, acc.at[src_v.at[j]], add=True)
                return carry

            lax.fori_loop(0, GCH, _chunk, 0)

    @pl.when(c == 0)
    def _():
        _run(item0_hbm)

    @pl.when(c == 1)
    def _():
        _run(item1_hbm)

    plsc.subcore_barrier()

    @pl.when(c == 0)
    def _():
        pltpu.sync_copy(acc.at[pl.ds(rbase, ROWS_PER_TILE)],
                        out_hbm.at[0, pl.ds(rbase, ROWS_PER_TILE)])

    @pl.when(c == 1)
    def _():
        pltpu.sync_copy(acc.at[pl.ds(rbase, ROWS_PER_TILE)],
                        out_hbm.at[1, pl.ds(rbase, ROWS_PER_TILE)])


_spmm = functools.partial(
    pl.kernel,
    mesh=plsc.VectorSubcoreMesh(core_axis_name="c", subcore_axis_name="s"),
    out_type=jax.ShapeDtypeStruct((2, N_PAD, DH), jnp.float32),
    scratch_types=[
        pltpu.VMEM((GCH, CHUNK), jnp.int32),
        pltpu.VMEM((GCH, CHUNK), jnp.int32),
        pltpu.VMEM((GCH, CHUNK), jnp.float32),
        pltpu.VMEM((CHUNK, DH), jnp.float32),
        pltpu.VMEM_SHARED((N_PAD, DH), jnp.float32),
    ],
)(_spmm_body)


def _w2_body(w_ref, o_ref):
    o_ref[...] = jnp.dot(w_ref[...], w_ref[...],
                         preferred_element_type=jnp.float32)


_w2 = pl.pallas_call(
    _w2_body,
    out_shape=jax.ShapeDtypeStruct((D, D), jnp.float32),
)


def _mm_body(agg_ref, u_ref, w2_ref, o_ref):
    a0 = agg_ref[0] * u_ref[...]
    a1 = agg_ref[1] * u_ref[...]
    o_ref[...] = (
        jnp.dot(a0, w2_ref[0], preferred_element_type=jnp.float32)
        + jnp.dot(a1, w2_ref[1], preferred_element_type=jnp.float32))


_mm = pl.pallas_call(
    _mm_body,
    grid=(N_U // RB,),
    in_specs=[
        pl.BlockSpec((2, RB, DH), lambda i: (0, i, 0)),
        pl.BlockSpec((RB, 1), lambda i: (i, 0)),
        pl.BlockSpec((2, DH, D), lambda i: (0, 0, 0)),
    ],
    out_specs=pl.BlockSpec((RB, D), lambda i: (i, 0)),
    out_shape=jax.ShapeDtypeStruct((N_U, D), jnp.float32),
)


def kernel(edge_index, edge_values, user_n_j, item_n_j, user_emb, item_emb,
           mean_weight):
    src = edge_index[0].astype(jnp.int32).reshape(N_TILES, NGROUP, GCH, CHUNK)
    dst = edge_index[1].astype(jnp.int32).reshape(N_TILES, NGROUP, GCH, CHUNK)
    ev = edge_values.reshape(N_TILES, NGROUP, GCH, CHUNK)
    item0 = item_emb[:, :DH]
    item1 = item_emb[:, DH:]
    agg = _spmm(dst, src, ev, item0, item1)[:, :N_U]
    w2 = _w2(mean_weight)
    return _mm(agg, user_n_j, w2.reshape(2, DH, D))


# speedup vs baseline: 5.9112x; 1.4114x over previous
import functools

import jax
import jax.numpy as jnp
from jax import lax
from jax.experimental import pallas as pl
from jax.experimental.pallas import tpu as pltpu
from jax.experimental.pallas import tpu_sc as plsc

N_U = 10000
N_I = 10000
E_TOT = 160000
D = 256
DH = 128
N_TILES = 16
CHUNK = 80
GCH = 25
NGROUP = 5
EDGES_PER_TILE = CHUNK * GCH * NGROUP
N_PAD = 10240
ROWS_PER_TILE = N_PAD // N_TILES
RB = 1000


def _spmm_body(dst_hbm, src_hbm, ev_hbm, item0_hbm, item1_hbm, out_hbm,
               dst_v, src_v, ev_v, rows0, rows1, acc, gsem0, gsem1):
    c = lax.axis_index("c")
    s = lax.axis_index("s")
    rbase = s * ROWS_PER_TILE

    zero16 = jnp.zeros((16,), jnp.float32)

    def _zb(i, carry):
        for j in range(DH // 16):
            rows0[i, pl.ds(j * 16, 16)] = zero16
        return carry

    lax.fori_loop(0, CHUNK, _zb, 0)
    for r in range(ROWS_PER_TILE // CHUNK):
        pltpu.sync_copy(rows0, acc.at[pl.ds(rbase + r * CHUNK, CHUNK)])
    plsc.subcore_barrier()

    def _scale(buf, j):
        def _edge16(gg, carry2):
            evv = ev_v[j, pl.ds(gg * 16, 16)]
            for lane in range(16):
                val = evv[lane]
                e = gg * 16 + lane
                for k in range(DH // 16):
                    sl = pl.ds(k * 16, 16)
                    buf[e, sl] = buf[e, sl] * val
            return carry2

        lax.fori_loop(0, CHUNK // 16, _edge16, 0)

    def _run(item_hbm):
        def _gstart(j, buf, sem):
            pltpu.make_async_copy(item_hbm.at[dst_v.at[j]], buf, sem).start()

        def _gwait(j, buf, sem):
            pltpu.make_async_copy(item_hbm.at[dst_v.at[j]], buf, sem).wait()

        for g in range(NGROUP):
            pltpu.sync_copy(dst_hbm.at[s, g], dst_v)
            pltpu.sync_copy(src_hbm.at[s, g], src_v)
            pltpu.sync_copy(ev_hbm.at[s, g], ev_v)

            _gstart(0, rows0, gsem0)

            def _pair(i, carry):
                j0 = i * 2
                _gwait(j0, rows0, gsem0)

                @pl.when(j0 + 1 < GCH)
                def _():
                    _gstart(j0 + 1, rows1, gsem1)

                _scale(rows0, j0)
                pltpu.sync_copy(rows0, acc.at[src_v.at[j0]], add=True)

                @pl.when(j0 + 1 < GCH)
                def _():
                    _gwait(j0 + 1, rows1, gsem1)

                    @pl.when(j0 + 2 < GCH)
                    def _():
                        _gstart(j0 + 2, rows0, gsem0)

                    _scale(rows1, j0 + 1)
                    pltpu.sync_copy(rows1, acc.at[src_v.at[j0 + 1]],
                                    add=True)

                return carry

            lax.fori_loop(0, (GCH + 1) // 2, _pair, 0)

    @pl.when(c == 0)
    def _():
        _run(item0_hbm)

    @pl.when(c == 1)
    def _():
        _run(item1_hbm)

    plsc.subcore_barrier()

    @pl.when(c == 0)
    def _():
        pltpu.sync_copy(acc.at[pl.ds(rbase, ROWS_PER_TILE)],
                        out_hbm.at[0, pl.ds(rbase, ROWS_PER_TILE)])

    @pl.when(c == 1)
    def _():
        pltpu.sync_copy(acc.at[pl.ds(rbase, ROWS_PER_TILE)],
                        out_hbm.at[1, pl.ds(rbase, ROWS_PER_TILE)])


_spmm = functools.partial(
    pl.kernel,
    mesh=plsc.VectorSubcoreMesh(core_axis_name="c", subcore_axis_name="s"),
    out_type=jax.ShapeDtypeStruct((2, N_PAD, DH), jnp.float32),
    scratch_types=[
        pltpu.VMEM((GCH, CHUNK), jnp.int32),
        pltpu.VMEM((GCH, CHUNK), jnp.int32),
        pltpu.VMEM((GCH, CHUNK), jnp.float32),
        pltpu.VMEM((CHUNK, DH), jnp.float32),
        pltpu.VMEM((CHUNK, DH), jnp.float32),
        pltpu.VMEM_SHARED((N_PAD, DH), jnp.float32),
        pltpu.SemaphoreType.DMA,
        pltpu.SemaphoreType.DMA,
    ],
)(_spmm_body)


def _w2_body(w_ref, o_ref):
    o_ref[...] = jnp.dot(w_ref[...], w_ref[...],
                         preferred_element_type=jnp.float32)


_w2 = pl.pallas_call(
    _w2_body,
    out_shape=jax.ShapeDtypeStruct((D, D), jnp.float32),
)


def _mm_body(agg_ref, u_ref, w2_ref, o_ref):
    a0 = agg_ref[0] * u_ref[...]
    a1 = agg_ref[1] * u_ref[...]
    o_ref[...] = (
        jnp.dot(a0, w2_ref[0], preferred_element_type=jnp.float32)
        + jnp.dot(a1, w2_ref[1], preferred_element_type=jnp.float32))


_mm = pl.pallas_call(
    _mm_body,
    grid=(N_U // RB,),
    in_specs=[
        pl.BlockSpec((2, RB, DH), lambda i: (0, i, 0)),
        pl.BlockSpec((RB, 1), lambda i: (i, 0)),
        pl.BlockSpec((2, DH, D), lambda i: (0, 0, 0)),
    ],
    out_specs=pl.BlockSpec((RB, D), lambda i: (i, 0)),
    out_shape=jax.ShapeDtypeStruct((N_U, D), jnp.float32),
)


def kernel(edge_index, edge_values, user_n_j, item_n_j, user_emb, item_emb,
           mean_weight):
    src = edge_index[0].astype(jnp.int32).reshape(N_TILES, NGROUP, GCH, CHUNK)
    dst = edge_index[1].astype(jnp.int32).reshape(N_TILES, NGROUP, GCH, CHUNK)
    ev = edge_values.reshape(N_TILES, NGROUP, GCH, CHUNK)
    item0 = item_emb[:, :DH]
    item1 = item_emb[:, DH:]
    agg = _spmm(dst, src, ev, item0, item1)[:, :N_U]
    w2 = _w2(mean_weight)
    return _mm(agg, user_n_j, w2.reshape(2, DH, D))
